# baseline (speedup 1.0000x reference)
import functools

import jax
import jax.numpy as jnp
from jax import lax
from jax.experimental import pallas as pl
from jax.experimental.pallas import tpu as pltpu
from jax.experimental.pallas import tpu_sc as plsc

N = 10000
E = 320000
D = 128
NC = 2
NS = 16
NW = NC * NS
EPW = E // NW
BLK = 80
NBLK = EPW // BLK
EPWP = 10240
NBLKP = EPWP // BLK
NSTG = 4
DEPTH = 4
IBS = NBLKP // NSTG
NP_ = 10240
RPT = NP_ // NS
NP2 = NP_ // 8
RPT2 = NP2 // NS
BR = 2000
GRID = N // BR

@functools.cache
def _sc_kernels():
    mesh = plsc.VectorSubcoreMesh(core_axis_name="c", subcore_axis_name="s")

    @functools.partial(
        pl.kernel,
        mesh=mesh,
        out_type=jax.ShapeDtypeStruct((NC * NP_, D), jnp.float32),
        scratch_types=[
            pltpu.VMEM((IBS, BLK), jnp.int32),
            pltpu.VMEM((BLK, D), jnp.float32),
            pltpu.VMEM_SHARED((NP_, D), jnp.float32),
            pltpu.SemaphoreType.DMA,
        ],
    )
    def deg_kernel(dst_hbm, ones_hbm, zeros_hbm, out, dsti, ones_v, acc_sh, sem_s):
        c = lax.axis_index("c")
        s = lax.axis_index("s")
        wid = c * NS + s
        pltpu.sync_copy(ones_hbm, ones_v)
        pltpu.sync_copy(zeros_hbm.at[pl.ds(s * RPT, RPT)],
                        acc_sh.at[pl.ds(s * RPT, RPT)])
        plsc.subcore_barrier()

        def stage(st, carry):
            pltpu.sync_copy(dst_hbm.at[wid * NSTG + st], dsti)

            def body(i, carry2):
                scs = [
                    pltpu.async_copy(ones_v, acc_sh.at[dsti.at[4 * i + k]],
                                     sem_s, add=True)
                    for k in range(4)
                ]
                for sc in scs:
                    sc.wait()
                return carry2

            lax.fori_loop(0, IBS // 4, body, 0)
            return carry

        lax.fori_loop(0, NSTG, stage, 0)
        plsc.subcore_barrier()
        pltpu.sync_copy(acc_sh.at[pl.ds(s * RPT, RPT)],
                        out.at[pl.ds(c * NP_ + s * RPT, RPT)])

    @functools.partial(
        pl.kernel,
        mesh=mesh,
        out_type=jax.ShapeDtypeStruct((NC * NP_, D), jnp.float32),
        scratch_types=(
            [pltpu.VMEM((BLK,), jnp.int32) for _ in range(2 * DEPTH)]
            + [pltpu.VMEM((BLK, D), jnp.float32) for _ in range(DEPTH)]
            + [pltpu.SemaphoreType.DMA for _ in range(DEPTH + 1)]
            + [pltpu.VMEM_SHARED((NP_, D), jnp.float32)]
        ),
    )
    def agg_kernel(h_hbm, src_hbm, dst_hbm, zeros_hbm, out, *rest):
        srcf = rest[0:2 * DEPTH:2]
        dstf = rest[1:2 * DEPTH:2]
        rows = rest[2 * DEPTH:3 * DEPTH]
        sems = rest[3 * DEPTH:4 * DEPTH]
        sem_s = rest[4 * DEPTH]
        acc_sh = rest[4 * DEPTH + 1]
        c = lax.axis_index("c")
        s = lax.axis_index("s")
        wid = c * NS + s
        pltpu.sync_copy(zeros_hbm.at[pl.ds(s * RPT, RPT)],
                        acc_sh.at[pl.ds(s * RPT, RPT)])
        plsc.subcore_barrier()
        base = wid * EPW

        def body(i, carry):
            gs = []
            for k in range(DEPTH):
                off = base + (DEPTH * i + k) * BLK
                pltpu.sync_copy(src_hbm.at[pl.ds(off, BLK)], srcf[k])
                pltpu.sync_copy(dst_hbm.at[pl.ds(off, BLK)], dstf[k])
                gs.append(pltpu.async_copy(h_hbm.at[srcf[k]], rows[k], sems[k]))
            scs = []
            for k in range(DEPTH):
                gs[k].wait()
                scs.append(pltpu.async_copy(rows[k], acc_sh.at[dstf[k]],
                                            sem_s, add=True))
            for sc in scs:
                sc.wait()
            return carry

        lax.fori_loop(0, NBLK // DEPTH, body, 0)
        for j in range(NBLK - NBLK % DEPTH, NBLK):
            off = base + j * BLK
            pltpu.sync_copy(src_hbm.at[pl.ds(off, BLK)], srcf[0])
            pltpu.sync_copy(dst_hbm.at[pl.ds(off, BLK)], dstf[0])
            pltpu.async_copy(h_hbm.at[srcf[0]], rows[0], sems[0]).wait()
            pltpu.sync_copy(rows[0], acc_sh.at[dstf[0]], add=True)
        plsc.subcore_barrier()
        pltpu.sync_copy(acc_sh.at[pl.ds(s * RPT, RPT)],
                        out.at[pl.ds(c * NP_ + s * RPT, RPT)])

    return deg_kernel, agg_kernel


def _mm1_body(x_ref, d0_ref, d1_ref, w_ref, h_ref, dis_ref):
    deg = d0_ref[...] + d1_ref[...] + 1.0
    dis = lax.rsqrt(jnp.maximum(deg, 1e-12))
    dis_ref[...] = dis
    h_ref[...] = jnp.dot(x_ref[...], w_ref[...],
                         preferred_element_type=jnp.float32) * dis


_mm1 = pl.pallas_call(
    _mm1_body,
    grid=(GRID,),
    in_specs=[
        pl.BlockSpec((BR, D), lambda i: (i, 0)),
        pl.BlockSpec((BR, 1), lambda i: (i, 0)),
        pl.BlockSpec((BR, 1), lambda i: (i, 0)),
        pl.BlockSpec((D, D), lambda i: (0, 0)),
    ],
    out_specs=[
        pl.BlockSpec((BR, D), lambda i: (i, 0)),
        pl.BlockSpec((BR, 1), lambda i: (i, 0)),
    ],
    out_shape=[
        jax.ShapeDtypeStruct((N, D), jnp.float32),
        jax.ShapeDtypeStruct((N, 1), jnp.float32),
    ],
)


def _layer_body(p0_ref, p1_ref, hp_ref, dis_ref, b_ref, w_ref, out_ref):
    dis = dis_ref[...]
    x = dis * (p0_ref[...] + p1_ref[...] + hp_ref[...]) + b_ref[...]
    x = jnp.maximum(x, 0.0)
    out_ref[...] = jnp.dot(x, w_ref[...],
                           preferred_element_type=jnp.float32) * dis


_layer = pl.pallas_call(
    _layer_body,
    grid=(GRID,),
    in_specs=[
        pl.BlockSpec((BR, D), lambda i: (i, 0)),
        pl.BlockSpec((BR, D), lambda i: (i, 0)),
        pl.BlockSpec((BR, D), lambda i: (i, 0)),
        pl.BlockSpec((BR, 1), lambda i: (i, 0)),
        pl.BlockSpec((1, D), lambda i: (0, 0)),
        pl.BlockSpec((D, D), lambda i: (0, 0)),
    ],
    out_specs=pl.BlockSpec((BR, D), lambda i: (i, 0)),
    out_shape=jax.ShapeDtypeStruct((N, D), jnp.float32),
)


def _final_body(p0_ref, p1_ref, hp_ref, dis_ref, b_ref,
                g1w_ref, g1b_ref, g2w_ref, g2b_ref, lng_ref, lnb_ref,
                out_ref, acc_ref):
    i = pl.program_id(0)
    x3 = dis_ref[...] * (p0_ref[...] + p1_ref[...] + hp_ref[...]) + b_ref[...]
    psum = jnp.sum(x3, axis=0, keepdims=True)

    @pl.when(i == 0)
    def _():
        acc_ref[...] = jnp.zeros_like(acc_ref)

    acc_ref[...] += psum

    @pl.when(i == GRID - 1)
    def _():
        g = acc_ref[...] * (1.0 / N)
        z1 = jnp.maximum(
            jnp.dot(g, g1w_ref[...], preferred_element_type=jnp.float32)
            + g1b_ref[...], 0.0)
        z2 = (jnp.dot(z1, g2w_ref[...], preferred_element_type=jnp.float32)
              + g2b_ref[...])
        mu = jnp.mean(z2, axis=-1, keepdims=True)
        zc = z2 - mu
        var = jnp.mean(zc * zc, axis=-1, keepdims=True)
        zn = zc * lax.rsqrt(var + 1e-5)
        out_ref[...] = zn * lng_ref[...] + lnb_ref[...]


_final = pl.pallas_call(
    _final_body,
    grid=(GRID,),
    in_specs=[
        pl.BlockSpec((BR, D), lambda i: (i, 0)),
        pl.BlockSpec((BR, D), lambda i: (i, 0)),
        pl.BlockSpec((BR, D), lambda i: (i, 0)),
        pl.BlockSpec((BR, 1), lambda i: (i, 0)),
        pl.BlockSpec((1, D), lambda i: (0, 0)),
        pl.BlockSpec((D, D), lambda i: (0, 0)),
        pl.BlockSpec((1, D), lambda i: (0, 0)),
        pl.BlockSpec((D, D), lambda i: (0, 0)),
        pl.BlockSpec((1, D), lambda i: (0, 0)),
        pl.BlockSpec((1, D), lambda i: (0, 0)),
        pl.BlockSpec((1, D), lambda i: (0, 0)),
    ],
    out_specs=pl.BlockSpec((1, D), lambda i: (0, 0)),
    out_shape=jax.ShapeDtypeStruct((1, D), jnp.float32),
    scratch_shapes=[pltpu.VMEM((1, D), jnp.float32)],
)


def kernel(node_features, edge_index, W1, b1, W2, b2, W3, b3,
           G1_W, G1_b, G2_W, G2_b, ln_gamma, ln_beta):
    ei = edge_index.astype(jnp.int32)

    def _chunked(flat, pad_value):
        a = flat.reshape(NW, EPW)
        a = jnp.pad(a, ((0, 0), (0, EPWP - EPW)), constant_values=pad_value)
        return a.reshape(NW * NSTG, IBS, BLK)

    srcf = ei[0]
    dstf = ei[1]
    dst = _chunked(ei[1], N)
    ones_pat = jnp.zeros((BLK, D), jnp.float32).at[:, 0].set(1.0)
    zeros_nd = jnp.zeros((NP_, D), jnp.float32)
    _deg_kernel, _agg_kernel = _sc_kernels()

    degp = _deg_kernel(dst, ones_pat, zeros_nd)
    d0 = degp[0:N, 0:1]
    d1 = degp[NP_:NP_ + N, 0:1]

    h1p, dis = _mm1(node_features, d0, d1, W1)

    a1 = _agg_kernel(h1p, srcf, dstf, zeros_nd)
    h2p = _layer(a1[:N], a1[NP_:NP_ + N], h1p, dis, b1.reshape(1, D), W2)
    a2 = _agg_kernel(h2p, srcf, dstf, zeros_nd)
    h3p = _layer(a2[:N], a2[NP_:NP_ + N], h2p, dis, b2.reshape(1, D), W3)
    a3 = _agg_kernel(h3p, srcf, dstf, zeros_nd)

    return _final(a3[:N], a3[NP_:NP_ + N], h3p, dis, b3.reshape(1, D),
                  G1_W, G1_b.reshape(1, D), G2_W, G2_b.reshape(1, D),
                  ln_gamma.reshape(1, D), ln_beta.reshape(1, D))


# speedup vs baseline: 1.2222x; 1.2222x over previous
import functools

import jax
import jax.numpy as jnp
from jax import lax
from jax.experimental import pallas as pl
from jax.experimental.pallas import tpu as pltpu
from jax.experimental.pallas import tpu_sc as plsc

N = 10000
E = 320000
D = 128
NC = 2
NS = 16
NW = NC * NS
EPW = E // NW
BLK = 80
NBLK = EPW // BLK
EPWP = 10240
NBLKP = EPWP // BLK
NSTG = 4
DEPTH = 4
IBS = NBLKP // NSTG
NP_ = 10240
RPT = NP_ // NS
NP2 = NP_ // 8
RPT2 = NP2 // NS
BR = 2000
GRID = N // BR

@functools.cache
def _sc_kernels():
    mesh = plsc.VectorSubcoreMesh(core_axis_name="c", subcore_axis_name="s")

    @functools.partial(
        pl.kernel,
        mesh=mesh,
        out_type=jax.ShapeDtypeStruct((NC * NP_, D), jnp.float32),
        scratch_types=[
            pltpu.VMEM((IBS, BLK), jnp.int32),
            pltpu.VMEM((BLK, D), jnp.float32),
            pltpu.VMEM_SHARED((NP_, D), jnp.float32),
            pltpu.SemaphoreType.DMA,
        ],
    )
    def deg_kernel(dst_hbm, ones_hbm, zeros_hbm, out, dsti, ones_v, acc_sh, sem_s):
        c = lax.axis_index("c")
        s = lax.axis_index("s")
        wid = c * NS + s
        pltpu.sync_copy(ones_hbm, ones_v)
        pltpu.sync_copy(zeros_hbm.at[pl.ds(s * RPT, RPT)],
                        acc_sh.at[pl.ds(s * RPT, RPT)])
        plsc.subcore_barrier()

        def stage(st, carry):
            pltpu.sync_copy(dst_hbm.at[wid * NSTG + st], dsti)

            def body(i, carry2):
                scs = [
                    pltpu.async_copy(ones_v, acc_sh.at[dsti.at[4 * i + k]],
                                     sem_s, add=True)
                    for k in range(4)
                ]
                for sc in scs:
                    sc.wait()
                return carry2

            lax.fori_loop(0, IBS // 4, body, 0)
            return carry

        lax.fori_loop(0, NSTG, stage, 0)
        plsc.subcore_barrier()
        pltpu.sync_copy(acc_sh.at[pl.ds(s * RPT, RPT)],
                        out.at[pl.ds(c * NP_ + s * RPT, RPT)])

    @functools.partial(
        pl.kernel,
        mesh=mesh,
        out_type=jax.ShapeDtypeStruct((NC * NP_, D), jnp.float32),
        scratch_types=(
            [pltpu.VMEM((2, BLK), jnp.int32) for _ in range(DEPTH)]
            + [pltpu.VMEM((BLK, D), jnp.float32) for _ in range(DEPTH)]
            + [pltpu.SemaphoreType.DMA for _ in range(DEPTH + 1)]
            + [pltpu.VMEM_SHARED((NP_, D), jnp.float32)]
        ),
    )
    def agg_kernel(h_hbm, esd_hbm, zeros_hbm, out, *rest):
        idxb = rest[0:DEPTH]
        rows = rest[DEPTH:2 * DEPTH]
        sems = rest[2 * DEPTH:3 * DEPTH]
        sem_s = rest[3 * DEPTH]
        acc_sh = rest[3 * DEPTH + 1]
        c = lax.axis_index("c")
        s = lax.axis_index("s")
        wid = c * NS + s
        pltpu.sync_copy(zeros_hbm.at[pl.ds(s * RPT, RPT)],
                        acc_sh.at[pl.ds(s * RPT, RPT)])
        plsc.subcore_barrier()
        bblk = wid * NBLK

        def body(i, carry):
            gs = []
            for k in range(DEPTH):
                pltpu.sync_copy(esd_hbm.at[bblk + DEPTH * i + k], idxb[k])
                gs.append(pltpu.async_copy(h_hbm.at[idxb[k].at[0]], rows[k],
                                           sems[k]))
            scs = []
            for k in range(DEPTH):
                gs[k].wait()
                scs.append(pltpu.async_copy(rows[k], acc_sh.at[idxb[k].at[1]],
                                            sem_s, add=True))
            for sc in scs:
                sc.wait()
            return carry

        lax.fori_loop(0, NBLK // DEPTH, body, 0)
        for j in range(NBLK - NBLK % DEPTH, NBLK):
            pltpu.sync_copy(esd_hbm.at[bblk + j], idxb[0])
            pltpu.async_copy(h_hbm.at[idxb[0].at[0]], rows[0], sems[0]).wait()
            pltpu.sync_copy(rows[0], acc_sh.at[idxb[0].at[1]], add=True)
        plsc.subcore_barrier()
        pltpu.sync_copy(acc_sh.at[pl.ds(s * RPT, RPT)],
                        out.at[pl.ds(c * NP_ + s * RPT, RPT)])

    return deg_kernel, agg_kernel


def _mm1_body(x_ref, d0_ref, d1_ref, w_ref, h_ref, dis_ref):
    deg = d0_ref[...] + d1_ref[...] + 1.0
    dis = lax.rsqrt(jnp.maximum(deg, 1e-12))
    dis_ref[...] = dis
    h_ref[...] = jnp.dot(x_ref[...], w_ref[...],
                         preferred_element_type=jnp.float32) * dis


_mm1 = pl.pallas_call(
    _mm1_body,
    grid=(GRID,),
    in_specs=[
        pl.BlockSpec((BR, D), lambda i: (i, 0)),
        pl.BlockSpec((BR, 1), lambda i: (i, 0)),
        pl.BlockSpec((BR, 1), lambda i: (i, 0)),
        pl.BlockSpec((D, D), lambda i: (0, 0)),
    ],
    out_specs=[
        pl.BlockSpec((BR, D), lambda i: (i, 0)),
        pl.BlockSpec((BR, 1), lambda i: (i, 0)),
    ],
    out_shape=[
        jax.ShapeDtypeStruct((N, D), jnp.float32),
        jax.ShapeDtypeStruct((N, 1), jnp.float32),
    ],
)


def _layer_body(p0_ref, p1_ref, hp_ref, dis_ref, b_ref, w_ref, out_ref):
    dis = dis_ref[...]
    x = dis * (p0_ref[...] + p1_ref[...] + hp_ref[...]) + b_ref[...]
    x = jnp.maximum(x, 0.0)
    out_ref[...] = jnp.dot(x, w_ref[...],
                           preferred_element_type=jnp.float32) * dis


_layer = pl.pallas_call(
    _layer_body,
    grid=(GRID,),
    in_specs=[
        pl.BlockSpec((BR, D), lambda i: (i, 0)),
        pl.BlockSpec((BR, D), lambda i: (i, 0)),
        pl.BlockSpec((BR, D), lambda i: (i, 0)),
        pl.BlockSpec((BR, 1), lambda i: (i, 0)),
        pl.BlockSpec((1, D), lambda i: (0, 0)),
        pl.BlockSpec((D, D), lambda i: (0, 0)),
    ],
    out_specs=pl.BlockSpec((BR, D), lambda i: (i, 0)),
    out_shape=jax.ShapeDtypeStruct((N, D), jnp.float32),
)


def _final_body(p0_ref, p1_ref, hp_ref, dis_ref, b_ref,
                g1w_ref, g1b_ref, g2w_ref, g2b_ref, lng_ref, lnb_ref,
                out_ref, acc_ref):
    i = pl.program_id(0)
    x3 = dis_ref[...] * (p0_ref[...] + p1_ref[...] + hp_ref[...]) + b_ref[...]
    psum = jnp.sum(x3, axis=0, keepdims=True)

    @pl.when(i == 0)
    def _():
        acc_ref[...] = jnp.zeros_like(acc_ref)

    acc_ref[...] += psum

    @pl.when(i == GRID - 1)
    def _():
        g = acc_ref[...] * (1.0 / N)
        z1 = jnp.maximum(
            jnp.dot(g, g1w_ref[...], preferred_element_type=jnp.float32)
            + g1b_ref[...], 0.0)
        z2 = (jnp.dot(z1, g2w_ref[...], preferred_element_type=jnp.float32)
              + g2b_ref[...])
        mu = jnp.mean(z2, axis=-1, keepdims=True)
        zc = z2 - mu
        var = jnp.mean(zc * zc, axis=-1, keepdims=True)
        zn = zc * lax.rsqrt(var + 1e-5)
        out_ref[...] = zn * lng_ref[...] + lnb_ref[...]


_final = pl.pallas_call(
    _final_body,
    grid=(GRID,),
    in_specs=[
        pl.BlockSpec((BR, D), lambda i: (i, 0)),
        pl.BlockSpec((BR, D), lambda i: (i, 0)),
        pl.BlockSpec((BR, D), lambda i: (i, 0)),
        pl.BlockSpec((BR, 1), lambda i: (i, 0)),
        pl.BlockSpec((1, D), lambda i: (0, 0)),
        pl.BlockSpec((D, D), lambda i: (0, 0)),
        pl.BlockSpec((1, D), lambda i: (0, 0)),
        pl.BlockSpec((D, D), lambda i: (0, 0)),
        pl.BlockSpec((1, D), lambda i: (0, 0)),
        pl.BlockSpec((1, D), lambda i: (0, 0)),
        pl.BlockSpec((1, D), lambda i: (0, 0)),
    ],
    out_specs=pl.BlockSpec((1, D), lambda i: (0, 0)),
    out_shape=jax.ShapeDtypeStruct((1, D), jnp.float32),
    scratch_shapes=[pltpu.VMEM((1, D), jnp.float32)],
)


def kernel(node_features, edge_index, W1, b1, W2, b2, W3, b3,
           G1_W, G1_b, G2_W, G2_b, ln_gamma, ln_beta):
    ei = edge_index.astype(jnp.int32)

    def _chunked(flat, pad_value):
        a = flat.reshape(NW, EPW)
        a = jnp.pad(a, ((0, 0), (0, EPWP - EPW)), constant_values=pad_value)
        return a.reshape(NW * NSTG, IBS, BLK)

    esd = (ei.reshape(2, NW, NBLK, BLK).transpose(1, 2, 0, 3)
           .reshape(NW * NBLK, 2, BLK))
    dst = _chunked(ei[1], N)
    ones_pat = jnp.zeros((BLK, D), jnp.float32).at[:, 0].set(1.0)
    zeros_nd = jnp.zeros((NP_, D), jnp.float32)
    _deg_kernel, _agg_kernel = _sc_kernels()

    degp = _deg_kernel(dst, ones_pat, zeros_nd)
    d0 = degp[0:N, 0:1]
    d1 = degp[NP_:NP_ + N, 0:1]

    h1p, dis = _mm1(node_features, d0, d1, W1)

    a1 = _agg_kernel(h1p, esd, zeros_nd)
    h2p = _layer(a1[:N], a1[NP_:NP_ + N], h1p, dis, b1.reshape(1, D), W2)
    a2 = _agg_kernel(h2p, esd, zeros_nd)
    h3p = _layer(a2[:N], a2[NP_:NP_ + N], h2p, dis, b2.reshape(1, D), W3)
    a3 = _agg_kernel(h3p, esd, zeros_nd)

    return _final(a3[:N], a3[NP_:NP_ + N], h3p, dis, b3.reshape(1, D),
                  G1_W, G1_b.reshape(1, D), G2_W, G2_b.reshape(1, D),
                  ln_gamma.reshape(1, D), ln_beta.reshape(1, D))
